# baseline (speedup 1.0000x reference)
import jax
import jax.numpy as jnp
from jax import lax
from jax.experimental import pallas as pl
from jax.experimental.pallas import tpu as pltpu
from jax.experimental.pallas import tpu_sc as plsc

NNZ = 160000
NSEG = 10000
NSEG_PAD = 10240
G = 128
NGROUPS = NNZ // G
NSUB = 16
NCORES = 2
NW = NSUB * NCORES
ROWS_PER_TILE = NSEG_PAD // NSUB
NB = 2

_MESH = dict(core_axis_name="c", subcore_axis_name="s")
_PARAMS = dict(
    mesh=plsc.VectorSubcoreMesh(**_MESH),
    compiler_params=pltpu.CompilerParams(use_tc_tiling_on_sc=False),
)


def _sc_scatter(x, ridx2, cidx2):
    w = x.shape[1]
    K, GS = (1, 64) if w >= 128 else (2, 128)
    ngr = NNZ // GS
    KG = K * GS
    ZR = min(KG, 128)

    def body(x_hbm, ridx_hbm, cidx_hbm, rowsum_hbm, colsum_hbm,
             table, xbufs, idxbufs, xsem, isem, scsem):
        cid = lax.axis_index("c")
        sid = lax.axis_index("s")
        lo = (ngr * sid) // NSUB
        hi = (ngr * (sid + 1)) // NSUB
        r0 = sid * ROWS_PER_TILE

        def zrow(r, carry):
            for k in range(w // 16):
                xbufs[0, r, pl.ds(k * 16, 16)] = jnp.zeros((16,), jnp.float32)
            return carry
        lax.fori_loop(0, ZR, zrow, 0)
        for piece in range(ROWS_PER_TILE // ZR):
            pltpu.sync_copy(xbufs.at[0, pl.ds(0, ZR)],
                            table.at[pl.ds(r0 + piece * ZR, ZR)])
        plsc.subcore_barrier()

        def do_scatter(idx_hbm):
            nsb = (hi - lo) // K

            def fire_in(s, b):
                g0 = lo + s * K
                pltpu.async_copy(x_hbm.at[pl.ds(g0 * GS, KG)], xbufs.at[b],
                                 xsem.at[b])
                pltpu.async_copy(idx_hbm.at[pl.ds(g0, K)], idxbufs.at[b],
                                 isem.at[b])

            def wait_in(b):
                pltpu.make_async_copy(x_hbm.at[pl.ds(0, KG)], xbufs.at[b],
                                      xsem.at[b]).wait()
                pltpu.make_async_copy(idx_hbm.at[pl.ds(0, K)], idxbufs.at[b],
                                      isem.at[b]).wait()

            def fire_scatter(b):
                for j in range(K):
                    pltpu.async_copy(xbufs.at[b, pl.ds(j * GS, GS)],
                                     table.at[idxbufs.at[b, j]],
                                     scsem.at[b], add=True)

            def wait_scatter(b):
                for j in range(K):
                    pltpu.make_async_copy(xbufs.at[b, pl.ds(j * GS, GS)],
                                          table.at[idxbufs.at[b, j]],
                                          scsem.at[b]).wait()

            @pl.when(nsb > 0)
            def _():
                fire_in(0, 0)

            def loop(s, carry):
                b = s % NB

                wait_in(b)

                @pl.when(s >= 1)
                def _():
                    wait_scatter((s - 1) % NB)

                @pl.when(s + 1 < nsb)
                def _():
                    fire_in(s + 1, (s + 1) % NB)

                fire_scatter(b)
                return carry
            lax.fori_loop(0, nsb, loop, 0)

            @pl.when(nsb >= 1)
            def _():
                wait_scatter((nsb - 1) % NB)

            def rem(g, carry):
                pltpu.sync_copy(x_hbm.at[pl.ds(g * GS, GS)],
                                xbufs.at[0, pl.ds(0, GS)])
                pltpu.sync_copy(idx_hbm.at[pl.ds(g, 1)],
                                idxbufs.at[0, pl.ds(0, 1)])
                pltpu.sync_copy(xbufs.at[0, pl.ds(0, GS)],
                                table.at[idxbufs.at[0, 0]], add=True)
                return carry
            lax.fori_loop(lo + nsb * K, hi, rem, 0)

        @pl.when(cid == 0)
        def _():
            do_scatter(ridx_hbm)

        @pl.when(cid == 1)
        def _():
            do_scatter(cidx_hbm)

        plsc.subcore_barrier()

        @pl.when(cid == 0)
        def _():
            pltpu.sync_copy(table.at[pl.ds(r0, ROWS_PER_TILE)],
                            rowsum_hbm.at[pl.ds(r0, ROWS_PER_TILE)])

        @pl.when(cid == 1)
        def _():
            pltpu.sync_copy(table.at[pl.ds(r0, ROWS_PER_TILE)],
                            colsum_hbm.at[pl.ds(r0, ROWS_PER_TILE)])

    tab_t = jax.ShapeDtypeStruct((NSEG_PAD, w), jnp.float32)
    f = pl.kernel(
        body,
        out_type=(tab_t, tab_t),
        scratch_types=[
            pltpu.VMEM_SHARED((NSEG_PAD, w), jnp.float32),
            pltpu.VMEM((NB, KG, w), jnp.float32),
            pltpu.VMEM((NB, K, GS), jnp.int32),
            pltpu.SemaphoreType.DMA((NB,)),
            pltpu.SemaphoreType.DMA((NB,)),
            pltpu.SemaphoreType.DMA((NB,)),
        ],
        **_PARAMS,
    )
    r2 = ridx2.reshape(ngr, GS)
    c2 = cidx2.reshape(ngr, GS)
    return f(x, r2, c2)


def _sc_gather_combine(rowtab, coltab, xw, ridx2, cidx2, *, lrelu):
    o = rowtab.shape[1]

    def body(rowtab_hbm, coltab_hbm, xw_hbm, ridx_hbm, cidx_hbm, out_hbm,
             rbufs, cbufs, xbufs, ridxbufs, cidxbufs,
             risem, cisem, gsem, osem):
        cid = lax.axis_index("c")
        sid = lax.axis_index("s")
        wid = cid * NSUB + sid
        lo = (NGROUPS * wid) // NW
        hi = (NGROUPS * (wid + 1)) // NW
        n = hi - lo

        def fire_idx(g, b):
            pltpu.async_copy(ridx_hbm.at[pl.ds(g, 1)], ridxbufs.at[b],
                             risem.at[b])
            pltpu.async_copy(cidx_hbm.at[pl.ds(g, 1)], cidxbufs.at[b],
                             cisem.at[b])

        def wait_idx(b):
            pltpu.make_async_copy(ridx_hbm.at[pl.ds(0, 1)], ridxbufs.at[b],
                                  risem.at[b]).wait()
            pltpu.make_async_copy(cidx_hbm.at[pl.ds(0, 1)], cidxbufs.at[b],
                                  cisem.at[b]).wait()

        def fire_gather(g, b):
            pltpu.async_copy(rowtab_hbm.at[ridxbufs.at[b, 0]], rbufs.at[b],
                             gsem.at[b])
            pltpu.async_copy(coltab_hbm.at[cidxbufs.at[b, 0]], cbufs.at[b],
                             gsem.at[b])
            pltpu.async_copy(xw_hbm.at[pl.ds(g * G, G)], xbufs.at[b],
                             gsem.at[b])

        def wait_gather(b):
            pltpu.make_async_copy(rowtab_hbm.at[ridxbufs.at[b, 0]],
                                  rbufs.at[b], gsem.at[b]).wait()
            pltpu.make_async_copy(coltab_hbm.at[cidxbufs.at[b, 0]],
                                  cbufs.at[b], gsem.at[b]).wait()
            pltpu.make_async_copy(xw_hbm.at[pl.ds(0, G)], xbufs.at[b],
                                  gsem.at[b]).wait()

        def combine(b):
            def vrow(r, carry):
                for k in range(o // 16):
                    sl = pl.ds(k * 16, 16)
                    a = xbufs[b, r, sl] + rbufs[b, r, sl] + cbufs[b, r, sl]
                    if lrelu:
                        a = jnp.maximum(a, a * jnp.float32(0.01))
                    xbufs[b, r, sl] = a
                return carry
            lax.fori_loop(0, G, vrow, 0)

        def fire_out(g, b):
            pltpu.async_copy(xbufs.at[b], out_hbm.at[pl.ds(g * G, G)],
                             osem.at[b])

        def wait_out(b):
            pltpu.make_async_copy(xbufs.at[b], out_hbm.at[pl.ds(0, G)],
                                  osem.at[b]).wait()

        @pl.when(n > 0)
        def _():
            fire_idx(lo, 0)

        def loop(i, carry):
            g = lo + i
            b = i % NB

            wait_idx(b)

            @pl.when(i >= NB)
            def _():
                wait_out(b)

            fire_gather(g, b)

            @pl.when(i >= 1)
            def _():
                bp = (i - 1) % NB
                wait_gather(bp)
                combine(bp)
                fire_out(g - 1, bp)

            @pl.when(g + 1 < hi)
            def _():
                fire_idx(g + 1, (i + 1) % NB)

            return carry
        lax.fori_loop(0, n, loop, 0)

        @pl.when(n > 0)
        def _():
            b = (n - 1) % NB
            wait_gather(b)
            combine(b)
            fire_out(hi - 1, b)
            wait_out(b)

            @pl.when(n > 1)
            def _():
                wait_out((n - 2) % NB)

    f = pl.kernel(
        body,
        out_type=jax.ShapeDtypeStruct((NNZ, o), jnp.float32),
        scratch_types=[
            pltpu.VMEM((NB, G, o), jnp.float32),
            pltpu.VMEM((NB, G, o), jnp.float32),
            pltpu.VMEM((NB, G, o), jnp.float32),
            pltpu.VMEM((NB, 1, G), jnp.int32),
            pltpu.VMEM((NB, 1, G), jnp.int32),
            pltpu.SemaphoreType.DMA((NB,)),
            pltpu.SemaphoreType.DMA((NB,)),
            pltpu.SemaphoreType.DMA((NB,)),
            pltpu.SemaphoreType.DMA((NB,)),
        ],
        **_PARAMS,
    )
    return f(rowtab, coltab, xw, ridx2, cidx2)


def _sc_gather_concat(rowtab, coltab, ridx2, cidx2):
    o = rowtab.shape[1]

    def body(rowtab_hbm, coltab_hbm, ridx_hbm, cidx_hbm, out_hbm,
             rbufs, cbufs, obufs, ridxbufs, cidxbufs,
             risem, cisem, gsem, osem):
        cid = lax.axis_index("c")
        sid = lax.axis_index("s")
        wid = cid * NSUB + sid
        lo = (NGROUPS * wid) // NW
        hi = (NGROUPS * (wid + 1)) // NW
        n = hi - lo

        def fire_idx(g, b):
            pltpu.async_copy(ridx_hbm.at[pl.ds(g, 1)], ridxbufs.at[b],
                             risem.at[b])
            pltpu.async_copy(cidx_hbm.at[pl.ds(g, 1)], cidxbufs.at[b],
                             cisem.at[b])

        def wait_idx(b):
            pltpu.make_async_copy(ridx_hbm.at[pl.ds(0, 1)], ridxbufs.at[b],
                                  risem.at[b]).wait()
            pltpu.make_async_copy(cidx_hbm.at[pl.ds(0, 1)], cidxbufs.at[b],
                                  cisem.at[b]).wait()

        def fire_gather(b):
            pltpu.async_copy(rowtab_hbm.at[ridxbufs.at[b, 0]], rbufs.at[b],
                             gsem.at[b])
            pltpu.async_copy(coltab_hbm.at[cidxbufs.at[b, 0]], cbufs.at[b],
                             gsem.at[b])

        def wait_gather(b):
            pltpu.make_async_copy(rowtab_hbm.at[ridxbufs.at[b, 0]],
                                  rbufs.at[b], gsem.at[b]).wait()
            pltpu.make_async_copy(coltab_hbm.at[cidxbufs.at[b, 0]],
                                  cbufs.at[b], gsem.at[b]).wait()

        def combine(b):
            def vrow(r, carry):
                for k in range(o // 16):
                    obufs[b, r, pl.ds(k * 16, 16)] = (
                        rbufs[b, r, pl.ds(k * 16, 16)])
                    obufs[b, r, pl.ds(o + k * 16, 16)] = (
                        cbufs[b, r, pl.ds(k * 16, 16)])
                return carry
            lax.fori_loop(0, G, vrow, 0)

        def fire_out(g, b):
            pltpu.async_copy(obufs.at[b], out_hbm.at[pl.ds(g * G, G)],
                             osem.at[b])

        def wait_out(b):
            pltpu.make_async_copy(obufs.at[b], out_hbm.at[pl.ds(0, G)],
                                  osem.at[b]).wait()

        @pl.when(n > 0)
        def _():
            fire_idx(lo, 0)

        def loop(i, carry):
            g = lo + i
            b = i % NB

            wait_idx(b)

            @pl.when(i >= NB)
            def _():
                wait_out(b)

            fire_gather(b)

            @pl.when(i >= 1)
            def _():
                bp = (i - 1) % NB
                wait_gather(bp)
                combine(bp)
                fire_out(g - 1, bp)

            @pl.when(g + 1 < hi)
            def _():
                fire_idx(g + 1, (i + 1) % NB)

            return carry
        lax.fori_loop(0, n, loop, 0)

        @pl.when(n > 0)
        def _():
            b = (n - 1) % NB
            wait_gather(b)
            combine(b)
            fire_out(hi - 1, b)
            wait_out(b)

            @pl.when(n > 1)
            def _():
                wait_out((n - 2) % NB)

    f = pl.kernel(
        body,
        out_type=jax.ShapeDtypeStruct((NNZ, 2 * o), jnp.float32),
        scratch_types=[
            pltpu.VMEM((NB, G, o), jnp.float32),
            pltpu.VMEM((NB, G, o), jnp.float32),
            pltpu.VMEM((NB, G, 2 * o), jnp.float32),
            pltpu.VMEM((NB, 1, G), jnp.int32),
            pltpu.VMEM((NB, 1, G), jnp.int32),
            pltpu.SemaphoreType.DMA((NB,)),
            pltpu.SemaphoreType.DMA((NB,)),
            pltpu.SemaphoreType.DMA((NB,)),
            pltpu.SemaphoreType.DMA((NB,)),
        ],
        **_PARAMS,
    )
    return f(rowtab, coltab, ridx2, cidx2)


def _tc_edges(x, WxT, blk=2000):
    n, w = x.shape
    o = WxT.shape[1]
    grid = n // blk

    def body(x_ref, w_ref, xw_ref, gsum_ref):
        xb = x_ref[...]
        xw_ref[...] = jnp.dot(xb, w_ref[...],
                              preferred_element_type=jnp.float32)

        @pl.when(pl.program_id(0) == 0)
        def _():
            gsum_ref[...] = jnp.zeros_like(gsum_ref)
        gsum_ref[...] += jnp.sum(xb, axis=0, keepdims=True)

    return pl.pallas_call(
        body,
        grid=(grid,),
        in_specs=[pl.BlockSpec((blk, w), lambda i: (i, 0)),
                  pl.BlockSpec((w, o), lambda i: (0, 0))],
        out_specs=[pl.BlockSpec((blk, o), lambda i: (i, 0)),
                   pl.BlockSpec((1, w), lambda i: (0, 0))],
        out_shape=[jax.ShapeDtypeStruct((n, o), jnp.float32),
                   jax.ShapeDtypeStruct((1, w), jnp.float32)],
    )(x, WxT)


def _tc_tables(rowsum, colsum, rcnt, ccnt, gsum, WrT, WcT, WbT, b, blk=2048):
    w = rowsum.shape[1]
    o = WrT.shape[1]
    grid = NSEG_PAD // blk

    def body(rs_ref, cs_ref, rc_ref, cc_ref, gs_ref, wr_ref, wc_ref, wb_ref,
             b_ref, rowtab_ref, coltab_ref):
        const = (jnp.dot(gs_ref[...] * jnp.float32(1.0 / NNZ), wb_ref[...],
                         preferred_element_type=jnp.float32) + b_ref[...])
        rmean = rs_ref[...] / jnp.maximum(rc_ref[...], 1.0)
        cmean = cs_ref[...] / jnp.maximum(cc_ref[...], 1.0)
        rowtab_ref[...] = jnp.dot(rmean, wr_ref[...],
                                  preferred_element_type=jnp.float32) + const
        coltab_ref[...] = jnp.dot(cmean, wc_ref[...],
                                  preferred_element_type=jnp.float32)

    return pl.pallas_call(
        body,
        grid=(grid,),
        in_specs=[pl.BlockSpec((blk, w), lambda i: (i, 0)),
                  pl.BlockSpec((blk, w), lambda i: (i, 0)),
                  pl.BlockSpec((blk, 1), lambda i: (i, 0)),
                  pl.BlockSpec((blk, 1), lambda i: (i, 0)),
                  pl.BlockSpec((1, w), lambda i: (0, 0)),
                  pl.BlockSpec((w, o), lambda i: (0, 0)),
                  pl.BlockSpec((w, o), lambda i: (0, 0)),
                  pl.BlockSpec((w, o), lambda i: (0, 0)),
                  pl.BlockSpec((1, o), lambda i: (0, 0))],
        out_specs=[pl.BlockSpec((blk, o), lambda i: (i, 0)),
                   pl.BlockSpec((blk, o), lambda i: (i, 0))],
        out_shape=[jax.ShapeDtypeStruct((NSEG_PAD, o), jnp.float32),
                   jax.ShapeDtypeStruct((NSEG_PAD, o), jnp.float32)],
    )(rowsum, colsum, rcnt, ccnt, gsum, WrT, WcT, WbT, b)


def _tc_means(rowsum, colsum, rcnt, ccnt, blk=2048):
    w = rowsum.shape[1]
    grid = NSEG_PAD // blk

    def body(rs_ref, cs_ref, rc_ref, cc_ref, rm_ref, cm_ref):
        rm_ref[...] = rs_ref[...] / jnp.maximum(rc_ref[...], 1.0)
        cm_ref[...] = cs_ref[...] / jnp.maximum(cc_ref[...], 1.0)

    return pl.pallas_call(
        body,
        grid=(grid,),
        in_specs=[pl.BlockSpec((blk, w), lambda i: (i, 0)),
                  pl.BlockSpec((blk, w), lambda i: (i, 0)),
                  pl.BlockSpec((blk, 1), lambda i: (i, 0)),
                  pl.BlockSpec((blk, 1), lambda i: (i, 0))],
        out_specs=[pl.BlockSpec((blk, w), lambda i: (i, 0)),
                   pl.BlockSpec((blk, w), lambda i: (i, 0))],
        out_shape=[jax.ShapeDtypeStruct((NSEG_PAD, w), jnp.float32),
                   jax.ShapeDtypeStruct((NSEG_PAD, w), jnp.float32)],
    )(rowsum, colsum, rcnt, ccnt)


def _prep_weights(W, b, d, dpad, opad):
    o = W.shape[0]
    parts = []
    for j in range(4):
        Wj = W[:, j * d:(j + 1) * d]
        Wj = jnp.pad(Wj, ((0, opad - o), (0, dpad - d))).T
        parts.append(Wj)
    bpad = jnp.pad(b, (0, opad - o)).reshape(1, opad)
    return parts[0], parts[1], parts[2], parts[3], bpad


def _layer(x, ridx2, cidx2, rcnt, ccnt, Wparts, *, lrelu):
    WxT, WrT, WcT, WbT, b = Wparts
    rowsum, colsum = _sc_scatter(x, ridx2, cidx2)
    xw, gsum = _tc_edges(x, WxT)
    rowtab, coltab = _tc_tables(rowsum, colsum, rcnt, ccnt, gsum,
                                WrT, WcT, WbT, b)
    return _sc_gather_combine(rowtab, coltab, xw, ridx2, cidx2, lrelu=lrelu)


def kernel(input, row_idx, col_idx,
           enc_W1, enc_b1, enc_W2, enc_b2, enc_W3, enc_b3,
           dec_W1, dec_b1, dec_W2, dec_b2, dec_W3, dec_b3):
    ridx2 = row_idx.reshape(NGROUPS, G)
    cidx2 = col_idx.reshape(NGROUPS, G)

    x0 = jnp.pad(input, ((0, 0), (0, 11))).at[:, 5].set(1.0)

    we1 = _prep_weights(enc_W1, enc_b1, 5, 16, 160)
    we2 = _prep_weights(enc_W2, enc_b2, 150, 160, 160)
    we3 = _prep_weights(enc_W3, enc_b3, 150, 160, 32)
    wd1 = _prep_weights(dec_W1, dec_b1, 64, 64, 160)
    wd2 = _prep_weights(dec_W2, dec_b2, 150, 160, 160)
    wd3 = _prep_weights(dec_W3, dec_b3, 150, 160, 16)

    rowsum1, colsum1 = _sc_scatter(x0, ridx2, cidx2)
    rcnt = rowsum1[:, 5:6]
    ccnt = colsum1[:, 5:6]
    xw1, gsum1 = _tc_edges(x0, we1[0])
    rowtab1, coltab1 = _tc_tables(rowsum1, colsum1, rcnt, ccnt, gsum1,
                                  we1[1], we1[2], we1[3], we1[4])
    h = _sc_gather_combine(rowtab1, coltab1, xw1, ridx2, cidx2, lrelu=True)

    h = _layer(h, ridx2, cidx2, rcnt, ccnt, we2, lrelu=True)
    encoded = _layer(h, ridx2, cidx2, rcnt, ccnt, we3, lrelu=False)

    prowsum, pcolsum = _sc_scatter(encoded, ridx2, cidx2)
    rowmean, colmean = _tc_means(prowsum, pcolsum, rcnt, ccnt)
    emb = _sc_gather_concat(rowmean, colmean, ridx2, cidx2)

    h = _layer(emb, ridx2, cidx2, rcnt, ccnt, wd1, lrelu=True)
    h = _layer(h, ridx2, cidx2, rcnt, ccnt, wd2, lrelu=True)
    out = _layer(h, ridx2, cidx2, rcnt, ccnt, wd3, lrelu=False)

    return out[:, :5]


# speedup vs baseline: 1.1294x; 1.1294x over previous
import jax
import jax.numpy as jnp
from jax import lax
from jax.experimental import pallas as pl
from jax.experimental.pallas import tpu as pltpu
from jax.experimental.pallas import tpu_sc as plsc

NNZ = 160000
NSEG = 10000
NSEG_PAD = 10240
G = 128
NGROUPS = NNZ // G
NSUB = 16
NCORES = 2
NW = NSUB * NCORES
ROWS_PER_TILE = NSEG_PAD // NSUB
NB = 2

_MESH = dict(core_axis_name="c", subcore_axis_name="s")
_PARAMS = dict(
    mesh=plsc.VectorSubcoreMesh(**_MESH),
    compiler_params=pltpu.CompilerParams(use_tc_tiling_on_sc=False),
)


def _sc_scatter(x, ridx2, cidx2):
    w = x.shape[1]
    K, GS = (1, 64) if w >= 128 else (2, 128)
    ngr = NNZ // GS
    KG = K * GS
    ZR = min(KG, 128)

    def body(x_hbm, ridx_hbm, cidx_hbm, rowsum_hbm, colsum_hbm,
             table, xbufs, idxbufs, xsem, isem, scsem):
        cid = lax.axis_index("c")
        sid = lax.axis_index("s")
        lo = (ngr * sid) // NSUB
        hi = (ngr * (sid + 1)) // NSUB
        r0 = sid * ROWS_PER_TILE

        def zrow(r, carry):
            for k in range(w // 16):
                xbufs[0, r, pl.ds(k * 16, 16)] = jnp.zeros((16,), jnp.float32)
            return carry
        lax.fori_loop(0, ZR, zrow, 0)
        for piece in range(ROWS_PER_TILE // ZR):
            pltpu.sync_copy(xbufs.at[0, pl.ds(0, ZR)],
                            table.at[pl.ds(r0 + piece * ZR, ZR)])
        plsc.subcore_barrier()

        def do_scatter(idx_hbm):
            nsb = (hi - lo) // K

            def fire_in(s, b):
                g0 = lo + s * K
                pltpu.async_copy(x_hbm.at[pl.ds(g0 * GS, KG)], xbufs.at[b],
                                 xsem.at[b])
                pltpu.async_copy(idx_hbm.at[pl.ds(g0, K)], idxbufs.at[b],
                                 isem.at[b])

            def wait_in(b):
                pltpu.make_async_copy(x_hbm.at[pl.ds(0, KG)], xbufs.at[b],
                                      xsem.at[b]).wait()
                pltpu.make_async_copy(idx_hbm.at[pl.ds(0, K)], idxbufs.at[b],
                                      isem.at[b]).wait()

            def fire_scatter(b):
                for j in range(K):
                    pltpu.async_copy(xbufs.at[b, pl.ds(j * GS, GS)],
                                     table.at[idxbufs.at[b, j]],
                                     scsem.at[b], add=True)

            def wait_scatter(b):
                for j in range(K):
                    pltpu.make_async_copy(xbufs.at[b, pl.ds(j * GS, GS)],
                                          table.at[idxbufs.at[b, j]],
                                          scsem.at[b]).wait()

            @pl.when(nsb > 0)
            def _():
                fire_in(0, 0)

            def loop(s, carry):
                b = s % NB

                wait_in(b)

                @pl.when(s >= 1)
                def _():
                    wait_scatter((s - 1) % NB)

                @pl.when(s + 1 < nsb)
                def _():
                    fire_in(s + 1, (s + 1) % NB)

                fire_scatter(b)
                return carry
            lax.fori_loop(0, nsb, loop, 0)

            @pl.when(nsb >= 1)
            def _():
                wait_scatter((nsb - 1) % NB)

            def rem(g, carry):
                pltpu.sync_copy(x_hbm.at[pl.ds(g * GS, GS)],
                                xbufs.at[0, pl.ds(0, GS)])
                pltpu.sync_copy(idx_hbm.at[pl.ds(g, 1)],
                                idxbufs.at[0, pl.ds(0, 1)])
                pltpu.sync_copy(xbufs.at[0, pl.ds(0, GS)],
                                table.at[idxbufs.at[0, 0]], add=True)
                return carry
            lax.fori_loop(lo + nsb * K, hi, rem, 0)

        @pl.when(cid == 0)
        def _():
            do_scatter(ridx_hbm)

        @pl.when(cid == 1)
        def _():
            do_scatter(cidx_hbm)

        plsc.subcore_barrier()

        @pl.when(cid == 0)
        def _():
            pltpu.sync_copy(table.at[pl.ds(r0, ROWS_PER_TILE)],
                            rowsum_hbm.at[pl.ds(r0, ROWS_PER_TILE)])

        @pl.when(cid == 1)
        def _():
            pltpu.sync_copy(table.at[pl.ds(r0, ROWS_PER_TILE)],
                            colsum_hbm.at[pl.ds(r0, ROWS_PER_TILE)])

    tab_t = jax.ShapeDtypeStruct((NSEG_PAD, w), jnp.float32)
    f = pl.kernel(
        body,
        out_type=(tab_t, tab_t),
        scratch_types=[
            pltpu.VMEM_SHARED((NSEG_PAD, w), jnp.float32),
            pltpu.VMEM((NB, KG, w), jnp.float32),
            pltpu.VMEM((NB, K, GS), jnp.int32),
            pltpu.SemaphoreType.DMA((NB,)),
            pltpu.SemaphoreType.DMA((NB,)),
            pltpu.SemaphoreType.DMA((NB,)),
        ],
        **_PARAMS,
    )
    r2 = ridx2.reshape(ngr, GS)
    c2 = cidx2.reshape(ngr, GS)
    return f(x, r2, c2)


def _sc_gather_combine(rowtab, coltab, xw, ridx2, cidx2, *, lrelu):
    o = rowtab.shape[1]

    def body(rowtab_hbm, coltab_hbm, xw_hbm, ridx_hbm, cidx_hbm, out_hbm,
             rbuf, cbuf, xbuf, ridxbuf, cidxbuf, sem):
        cid = lax.axis_index("c")
        sid = lax.axis_index("s")
        wid = cid * NSUB + sid
        lo = (NGROUPS * wid) // NW
        hi = (NGROUPS * (wid + 1)) // NW

        def step(g, carry):
            pltpu.sync_copy(ridx_hbm.at[pl.ds(g, 1)], ridxbuf)
            pltpu.sync_copy(cidx_hbm.at[pl.ds(g, 1)], cidxbuf)
            pltpu.async_copy(rowtab_hbm.at[ridxbuf.at[0]], rbuf, sem).wait()
            pltpu.async_copy(coltab_hbm.at[cidxbuf.at[0]], cbuf, sem).wait()
            pltpu.sync_copy(xw_hbm.at[pl.ds(g * G, G)], xbuf)

            def vrow(r, carry2):
                for k in range(o // 16):
                    sl = pl.ds(k * 16, 16)
                    a = xbuf[r, sl] + rbuf[r, sl] + cbuf[r, sl]
                    if lrelu:
                        a = jnp.maximum(a, a * jnp.float32(0.01))
                    xbuf[r, sl] = a
                return carry2
            lax.fori_loop(0, G, vrow, 0)
            pltpu.sync_copy(xbuf, out_hbm.at[pl.ds(g * G, G)])
            return carry
        lax.fori_loop(lo, hi, step, 0)

    f = pl.kernel(
        body,
        out_type=jax.ShapeDtypeStruct((NNZ, o), jnp.float32),
        scratch_types=[
            pltpu.VMEM((G, o), jnp.float32),
            pltpu.VMEM((G, o), jnp.float32),
            pltpu.VMEM((G, o), jnp.float32),
            pltpu.VMEM((1, G), jnp.int32),
            pltpu.VMEM((1, G), jnp.int32),
            pltpu.SemaphoreType.DMA,
        ],
        **_PARAMS,
    )
    return f(rowtab, coltab, xw, ridx2, cidx2)


def _sc_gather_concat(rowtab, coltab, ridx2, cidx2):
    o = rowtab.shape[1]

    def body(rowtab_hbm, coltab_hbm, ridx_hbm, cidx_hbm, out_hbm,
             rbuf, cbuf, obuf, ridxbuf, cidxbuf, sem):
        cid = lax.axis_index("c")
        sid = lax.axis_index("s")
        wid = cid * NSUB + sid
        lo = (NGROUPS * wid) // NW
        hi = (NGROUPS * (wid + 1)) // NW

        def step(g, carry):
            pltpu.sync_copy(ridx_hbm.at[pl.ds(g, 1)], ridxbuf)
            pltpu.sync_copy(cidx_hbm.at[pl.ds(g, 1)], cidxbuf)
            pltpu.async_copy(rowtab_hbm.at[ridxbuf.at[0]], rbuf, sem).wait()
            pltpu.async_copy(coltab_hbm.at[cidxbuf.at[0]], cbuf, sem).wait()

            def vrow(r, carry2):
                for k in range(o // 16):
                    obuf[r, pl.ds(k * 16, 16)] = rbuf[r, pl.ds(k * 16, 16)]
                    obuf[r, pl.ds(o + k * 16, 16)] = cbuf[r, pl.ds(k * 16, 16)]
                return carry2
            lax.fori_loop(0, G, vrow, 0)
            pltpu.sync_copy(obuf, out_hbm.at[pl.ds(g * G, G)])
            return carry
        lax.fori_loop(lo, hi, step, 0)

    f = pl.kernel(
        body,
        out_type=jax.ShapeDtypeStruct((NNZ, 2 * o), jnp.float32),
        scratch_types=[
            pltpu.VMEM((G, o), jnp.float32),
            pltpu.VMEM((G, o), jnp.float32),
            pltpu.VMEM((G, 2 * o), jnp.float32),
            pltpu.VMEM((1, G), jnp.int32),
            pltpu.VMEM((1, G), jnp.int32),
            pltpu.SemaphoreType.DMA,
        ],
        **_PARAMS,
    )
    return f(rowtab, coltab, ridx2, cidx2)


def _tc_edges(x, WxT, blk=2000):
    n, w = x.shape
    o = WxT.shape[1]
    grid = n // blk

    def body(x_ref, w_ref, xw_ref, gsum_ref):
        xb = x_ref[...]
        xw_ref[...] = jnp.dot(xb, w_ref[...],
                              preferred_element_type=jnp.float32)

        @pl.when(pl.program_id(0) == 0)
        def _():
            gsum_ref[...] = jnp.zeros_like(gsum_ref)
        gsum_ref[...] += jnp.sum(xb, axis=0, keepdims=True)

    return pl.pallas_call(
        body,
        grid=(grid,),
        in_specs=[pl.BlockSpec((blk, w), lambda i: (i, 0)),
                  pl.BlockSpec((w, o), lambda i: (0, 0))],
        out_specs=[pl.BlockSpec((blk, o), lambda i: (i, 0)),
                   pl.BlockSpec((1, w), lambda i: (0, 0))],
        out_shape=[jax.ShapeDtypeStruct((n, o), jnp.float32),
                   jax.ShapeDtypeStruct((1, w), jnp.float32)],
    )(x, WxT)


def _tc_tables(rowsum, colsum, rcnt, ccnt, gsum, WrT, WcT, WbT, b, blk=2048):
    w = rowsum.shape[1]
    o = WrT.shape[1]
    grid = NSEG_PAD // blk

    def body(rs_ref, cs_ref, rc_ref, cc_ref, gs_ref, wr_ref, wc_ref, wb_ref,
             b_ref, rowtab_ref, coltab_ref):
        const = (jnp.dot(gs_ref[...] * jnp.float32(1.0 / NNZ), wb_ref[...],
                         preferred_element_type=jnp.float32) + b_ref[...])
        rmean = rs_ref[...] / jnp.maximum(rc_ref[...], 1.0)
        cmean = cs_ref[...] / jnp.maximum(cc_ref[...], 1.0)
        rowtab_ref[...] = jnp.dot(rmean, wr_ref[...],
                                  preferred_element_type=jnp.float32) + const
        coltab_ref[...] = jnp.dot(cmean, wc_ref[...],
                                  preferred_element_type=jnp.float32)

    return pl.pallas_call(
        body,
        grid=(grid,),
        in_specs=[pl.BlockSpec((blk, w), lambda i: (i, 0)),
                  pl.BlockSpec((blk, w), lambda i: (i, 0)),
                  pl.BlockSpec((blk, 1), lambda i: (i, 0)),
                  pl.BlockSpec((blk, 1), lambda i: (i, 0)),
                  pl.BlockSpec((1, w), lambda i: (0, 0)),
                  pl.BlockSpec((w, o), lambda i: (0, 0)),
                  pl.BlockSpec((w, o), lambda i: (0, 0)),
                  pl.BlockSpec((w, o), lambda i: (0, 0)),
                  pl.BlockSpec((1, o), lambda i: (0, 0))],
        out_specs=[pl.BlockSpec((blk, o), lambda i: (i, 0)),
                   pl.BlockSpec((blk, o), lambda i: (i, 0))],
        out_shape=[jax.ShapeDtypeStruct((NSEG_PAD, o), jnp.float32),
                   jax.ShapeDtypeStruct((NSEG_PAD, o), jnp.float32)],
    )(rowsum, colsum, rcnt, ccnt, gsum, WrT, WcT, WbT, b)


def _tc_means(rowsum, colsum, rcnt, ccnt, blk=2048):
    w = rowsum.shape[1]
    grid = NSEG_PAD // blk

    def body(rs_ref, cs_ref, rc_ref, cc_ref, rm_ref, cm_ref):
        rm_ref[...] = rs_ref[...] / jnp.maximum(rc_ref[...], 1.0)
        cm_ref[...] = cs_ref[...] / jnp.maximum(cc_ref[...], 1.0)

    return pl.pallas_call(
        body,
        grid=(grid,),
        in_specs=[pl.BlockSpec((blk, w), lambda i: (i, 0)),
                  pl.BlockSpec((blk, w), lambda i: (i, 0)),
                  pl.BlockSpec((blk, 1), lambda i: (i, 0)),
                  pl.BlockSpec((blk, 1), lambda i: (i, 0))],
        out_specs=[pl.BlockSpec((blk, w), lambda i: (i, 0)),
                   pl.BlockSpec((blk, w), lambda i: (i, 0))],
        out_shape=[jax.ShapeDtypeStruct((NSEG_PAD, w), jnp.float32),
                   jax.ShapeDtypeStruct((NSEG_PAD, w), jnp.float32)],
    )(rowsum, colsum, rcnt, ccnt)


def _prep_weights(W, b, d, dpad, opad):
    o = W.shape[0]
    parts = []
    for j in range(4):
        Wj = W[:, j * d:(j + 1) * d]
        Wj = jnp.pad(Wj, ((0, opad - o), (0, dpad - d))).T
        parts.append(Wj)
    bpad = jnp.pad(b, (0, opad - o)).reshape(1, opad)
    return parts[0], parts[1], parts[2], parts[3], bpad


def _layer(x, ridx2, cidx2, rcnt, ccnt, Wparts, *, lrelu):
    WxT, WrT, WcT, WbT, b = Wparts
    rowsum, colsum = _sc_scatter(x, ridx2, cidx2)
    xw, gsum = _tc_edges(x, WxT)
    rowtab, coltab = _tc_tables(rowsum, colsum, rcnt, ccnt, gsum,
                                WrT, WcT, WbT, b)
    return _sc_gather_combine(rowtab, coltab, xw, ridx2, cidx2, lrelu=lrelu)


def kernel(input, row_idx, col_idx,
           enc_W1, enc_b1, enc_W2, enc_b2, enc_W3, enc_b3,
           dec_W1, dec_b1, dec_W2, dec_b2, dec_W3, dec_b3):
    ridx2 = row_idx.reshape(NGROUPS, G)
    cidx2 = col_idx.reshape(NGROUPS, G)

    x0 = jnp.concatenate(
        [input, jnp.ones((NNZ, 1), jnp.float32),
         jnp.zeros((NNZ, 10), jnp.float32)], axis=1)

    we1 = _prep_weights(enc_W1, enc_b1, 5, 16, 160)
    we2 = _prep_weights(enc_W2, enc_b2, 150, 160, 160)
    we3 = _prep_weights(enc_W3, enc_b3, 150, 160, 32)
    wd1 = _prep_weights(dec_W1, dec_b1, 64, 64, 160)
    wd2 = _prep_weights(dec_W2, dec_b2, 150, 160, 160)
    wd3 = _prep_weights(dec_W3, dec_b3, 150, 160, 16)

    rowsum1, colsum1 = _sc_scatter(x0, ridx2, cidx2)
    rcnt = rowsum1[:, 5:6]
    ccnt = colsum1[:, 5:6]
    xw1, gsum1 = _tc_edges(x0, we1[0])
    rowtab1, coltab1 = _tc_tables(rowsum1, colsum1, rcnt, ccnt, gsum1,
                                  we1[1], we1[2], we1[3], we1[4])
    h = _sc_gather_combine(rowtab1, coltab1, xw1, ridx2, cidx2, lrelu=True)

    h = _layer(h, ridx2, cidx2, rcnt, ccnt, we2, lrelu=True)
    encoded = _layer(h, ridx2, cidx2, rcnt, ccnt, we3, lrelu=False)

    prowsum, pcolsum = _sc_scatter(encoded, ridx2, cidx2)
    rowmean, colmean = _tc_means(prowsum, pcolsum, rcnt, ccnt)
    emb = _sc_gather_concat(rowmean, colmean, ridx2, cidx2)

    h = _layer(emb, ridx2, cidx2, rcnt, ccnt, wd1, lrelu=True)
    h = _layer(h, ridx2, cidx2, rcnt, ccnt, wd2, lrelu=True)
    out = _layer(h, ridx2, cidx2, rcnt, ccnt, wd3, lrelu=False)

    return out[:, :5]
